# baseline (speedup 1.0000x reference)
import functools

import jax
import jax.numpy as jnp
from jax import lax
from jax.experimental import pallas as pl
from jax.experimental.pallas import tpu as pltpu
from jax.experimental.pallas import tpu_sc as plsc

_N = 10000
_E = 320000
_D = 128
_NC = 2
_NS = 16
_NW = _NC * _NS
_EPT = _E // _NW
_C = 80
_NCHUNK = _EPT // _C
_RPT = 624
_RTAIL = _N - _RPT * _NS

_mesh = plsc.VectorSubcoreMesh(core_axis_name="c", subcore_axis_name="s")


@functools.partial(
    pl.kernel,
    out_type=jax.ShapeDtypeStruct((_NC * _N, _D), jnp.float32),
    mesh=_mesh,
    scratch_types=[
        pltpu.VMEM((_EPT,), jnp.int32),
        pltpu.VMEM((_NCHUNK, _C), jnp.int32),
        pltpu.VMEM((_C, _D), jnp.float32),
        pltpu.VMEM((_C, _D), jnp.float32),
        pltpu.VMEM_SHARED((_N, _D), jnp.float32),
        pltpu.SemaphoreType.DMA,
        pltpu.SemaphoreType.DMA,
        pltpu.SemaphoreType.DMA,
        pltpu.SemaphoreType.DMA,
    ],
)
def _sc_segment_sum(x_hbm, src_hbm, dst_hbm, zeros_hbm, out_hbm,
                    src_v, dst_v, rows0, rows1, acc, gs0, gs1, ss0, ss1):
    cid = lax.axis_index("c")
    sid = lax.axis_index("s")
    wid = sid * _NC + cid
    row_lo = sid * _RPT
    pltpu.sync_copy(src_hbm.at[pl.ds(wid * _EPT, _EPT)], src_v)
    pltpu.sync_copy(dst_hbm.at[wid], dst_v)
    pltpu.sync_copy(zeros_hbm.at[pl.ds(row_lo, _RPT)],
                    acc.at[pl.ds(row_lo, _RPT)])

    @pl.when(sid == _NS - 1)
    def _zero_tail():
        pltpu.sync_copy(zeros_hbm.at[pl.ds(_RPT * _NS, _RTAIL)],
                        acc.at[pl.ds(_RPT * _NS, _RTAIL)])

    plsc.subcore_barrier()

    def _src_chunk(i):
        return src_v.at[pl.ds(i * _C, _C)]

    pltpu.async_copy(x_hbm.at[_src_chunk(0)], rows0, gs0)
    npair = _NCHUNK // 2

    def _lin_chunk(i):
        return x_hbm.at[pl.ds(i * _C, _C)]

    def body(p, carry):
        a = 2 * p
        pltpu.make_async_copy(_lin_chunk(a), rows0, gs0).wait()
        pltpu.async_copy(_lin_chunk(a + 1), rows1, gs1)
        pltpu.make_async_copy(_lin_chunk(a + 1), rows1, gs1).wait()
        pltpu.async_copy(_lin_chunk(a + 2), rows0, gs0)
        return carry

    lax.fori_loop(0, npair, body, 0)
    last = _NCHUNK - 1
    pltpu.make_async_copy(x_hbm.at[_src_chunk(last)], rows0, gs0).wait()
    pltpu.async_copy(rows0, acc.at[dst_v.at[last]], ss0, add=True)
    pltpu.make_async_copy(rows0, acc.at[dst_v.at[0]], ss0).wait()
    plsc.subcore_barrier()
    pltpu.sync_copy(acc.at[pl.ds(row_lo, _RPT)],
                    out_hbm.at[pl.ds(cid * _N + row_lo, _RPT)])

    @pl.when(sid == _NS - 1)
    def _write_tail():
        pltpu.sync_copy(acc.at[pl.ds(_RPT * _NS, _RTAIL)],
                        out_hbm.at[pl.ds(cid * _N + _RPT * _NS, _RTAIL)])


def _fused_linear(p0, p1, x, w_rel, w_root, b2d, relu):
    nb = 25
    bs = _N // nb

    def body(p0_ref, p1_ref, x_ref, wrel_ref, wroot_ref, b_ref, o_ref):
        agg = p0_ref[...] + p1_ref[...]
        r = jnp.dot(agg, wrel_ref[...], preferred_element_type=jnp.float32)
        r = r + jnp.dot(x_ref[...], wroot_ref[...],
                        preferred_element_type=jnp.float32)
        r = r + b_ref[...]
        if relu:
            r = jnp.maximum(r, 0.0)
        o_ref[...] = r

    return pl.pallas_call(
        body,
        grid=(nb,),
        in_specs=[
            pl.BlockSpec((bs, _D), lambda i: (i, 0)),
            pl.BlockSpec((bs, _D), lambda i: (i, 0)),
            pl.BlockSpec((bs, _D), lambda i: (i, 0)),
            pl.BlockSpec((_D, _D), lambda i: (0, 0)),
            pl.BlockSpec((_D, _D), lambda i: (0, 0)),
            pl.BlockSpec((1, _D), lambda i: (0, 0)),
        ],
        out_specs=pl.BlockSpec((bs, _D), lambda i: (i, 0)),
        out_shape=jax.ShapeDtypeStruct((_N, _D), jnp.float32),
    )(p0, p1, x, w_rel, w_root, b2d)


def kernel(x, edge_index, W1_rel, b1, W1_root, W2_rel, b2, W2_root):
    src = edge_index[0]
    dst = edge_index[1].reshape(_NW, _NCHUNK, _C)
    zeros = jnp.zeros((_N, _D), jnp.float32)
    p = _sc_segment_sum(x, src, dst, zeros)
    h = _fused_linear(p[:_N], p[_N:], x, W1_rel, W1_root,
                      b1.reshape(1, _D), relu=True)
    p2 = _sc_segment_sum(h, src, dst, zeros)
    out = _fused_linear(p2[:_N], p2[_N:], h, W2_rel, W2_root,
                        b2.reshape(1, _D), relu=False)
    return out


# speedup vs baseline: 1.0436x; 1.0436x over previous
import functools

import jax
import jax.numpy as jnp
from jax import lax
from jax.experimental import pallas as pl
from jax.experimental.pallas import tpu as pltpu
from jax.experimental.pallas import tpu_sc as plsc

_N = 10000
_E = 320000
_D = 128
_NC = 2
_NS = 16
_NW = _NC * _NS
_EPT = _E // _NW
_C = 80
_NCHUNK = _EPT // _C
_RPT = 624
_RTAIL = _N - _RPT * _NS

_mesh = plsc.VectorSubcoreMesh(core_axis_name="c", subcore_axis_name="s")


@functools.partial(
    pl.kernel,
    out_type=jax.ShapeDtypeStruct((_NC * _N, _D), jnp.float32),
    mesh=_mesh,
    scratch_types=[
        pltpu.VMEM((_EPT,), jnp.int32),
        pltpu.VMEM((_NCHUNK, _C), jnp.int32),
        pltpu.VMEM((_C, _D), jnp.float32),
        pltpu.VMEM((_C, _D), jnp.float32),
        pltpu.VMEM_SHARED((_N, _D), jnp.float32),
        pltpu.SemaphoreType.DMA,
        pltpu.SemaphoreType.DMA,
        pltpu.SemaphoreType.DMA,
        pltpu.SemaphoreType.DMA,
        pltpu.SemaphoreType.DMA,
        pltpu.SemaphoreType.DMA,
    ],
)
def _sc_segment_sum(x_hbm, src_hbm, dst_hbm, zeros_hbm, out_hbm,
                    src_v, dst_v, rows0, rows1, acc,
                    g00, g01, g10, g11, ss0, ss1):
    cid = lax.axis_index("c")
    sid = lax.axis_index("s")
    wid = sid * _NC + cid
    row_lo = sid * _RPT
    pltpu.sync_copy(src_hbm.at[pl.ds(wid * _EPT, _EPT)], src_v)
    pltpu.sync_copy(dst_hbm.at[wid], dst_v)
    pltpu.sync_copy(zeros_hbm.at[pl.ds(row_lo, _RPT)],
                    acc.at[pl.ds(row_lo, _RPT)])

    @pl.when(sid == _NS - 1)
    def _zero_tail():
        pltpu.sync_copy(zeros_hbm.at[pl.ds(_RPT * _NS, _RTAIL)],
                        acc.at[pl.ds(_RPT * _NS, _RTAIL)])

    plsc.subcore_barrier()

    _H = _C // 2

    def _gather(k, h, buf, sem):
        idx = src_v.at[pl.ds(k * _C + h * _H, _H)]
        return pltpu.async_copy(x_hbm.at[idx], buf.at[pl.ds(h * _H, _H)], sem)

    def _gwait(k, h, buf, sem):
        idx = src_v.at[pl.ds(k * _C + h * _H, _H)]
        pltpu.make_async_copy(x_hbm.at[idx], buf.at[pl.ds(h * _H, _H)],
                              sem).wait()

    _gather(0, 0, rows0, g00)
    _gather(0, 1, rows0, g01)
    _gather(1, 0, rows1, g10)
    _gather(1, 1, rows1, g11)
    npair = _NCHUNK // 2

    def body(p, carry):
        a = 2 * p
        _gwait(a, 0, rows0, g00)
        _gwait(a, 1, rows0, g01)
        pltpu.async_copy(rows0, acc.at[dst_v.at[a]], ss0, add=True)
        _gwait(a + 1, 0, rows1, g10)
        _gwait(a + 1, 1, rows1, g11)
        pltpu.async_copy(rows1, acc.at[dst_v.at[a + 1]], ss1, add=True)
        pltpu.make_async_copy(rows0, acc.at[dst_v.at[a]], ss0).wait()
        _gather(a + 2, 0, rows0, g00)
        _gather(a + 2, 1, rows0, g01)

        @pl.when(p < npair - 1)
        def _refill1():
            pltpu.make_async_copy(rows1, acc.at[dst_v.at[a]], ss1).wait()
            _gather(a + 3, 0, rows1, g10)
            _gather(a + 3, 1, rows1, g11)

        return carry

    lax.fori_loop(0, npair, body, 0)
    last = _NCHUNK - 1
    _gwait(last, 0, rows0, g00)
    _gwait(last, 1, rows0, g01)
    pltpu.async_copy(rows0, acc.at[dst_v.at[last]], ss0, add=True)
    pltpu.make_async_copy(rows1, acc.at[dst_v.at[0]], ss1).wait()
    pltpu.make_async_copy(rows0, acc.at[dst_v.at[0]], ss0).wait()
    plsc.subcore_barrier()
    pltpu.sync_copy(acc.at[pl.ds(row_lo, _RPT)],
                    out_hbm.at[pl.ds(cid * _N + row_lo, _RPT)])

    @pl.when(sid == _NS - 1)
    def _write_tail():
        pltpu.sync_copy(acc.at[pl.ds(_RPT * _NS, _RTAIL)],
                        out_hbm.at[pl.ds(cid * _N + _RPT * _NS, _RTAIL)])


def _fused_linear(p0, p1, x, w_rel, w_root, b2d, relu):
    nb = 25
    bs = _N // nb

    def body(p0_ref, p1_ref, x_ref, wrel_ref, wroot_ref, b_ref, o_ref):
        agg = p0_ref[...] + p1_ref[...]
        r = jnp.dot(agg, wrel_ref[...], preferred_element_type=jnp.float32)
        r = r + jnp.dot(x_ref[...], wroot_ref[...],
                        preferred_element_type=jnp.float32)
        r = r + b_ref[...]
        if relu:
            r = jnp.maximum(r, 0.0)
        o_ref[...] = r

    return pl.pallas_call(
        body,
        grid=(nb,),
        in_specs=[
            pl.BlockSpec((bs, _D), lambda i: (i, 0)),
            pl.BlockSpec((bs, _D), lambda i: (i, 0)),
            pl.BlockSpec((bs, _D), lambda i: (i, 0)),
            pl.BlockSpec((_D, _D), lambda i: (0, 0)),
            pl.BlockSpec((_D, _D), lambda i: (0, 0)),
            pl.BlockSpec((1, _D), lambda i: (0, 0)),
        ],
        out_specs=pl.BlockSpec((bs, _D), lambda i: (i, 0)),
        out_shape=jax.ShapeDtypeStruct((_N, _D), jnp.float32),
    )(p0, p1, x, w_rel, w_root, b2d)


def kernel(x, edge_index, W1_rel, b1, W1_root, W2_rel, b2, W2_root):
    src = edge_index[0]
    dst = edge_index[1].reshape(_NW, _NCHUNK, _C)
    zeros = jnp.zeros((_N, _D), jnp.float32)
    p = _sc_segment_sum(x, src, dst, zeros)
    h = _fused_linear(p[:_N], p[_N:], x, W1_rel, W1_root,
                      b1.reshape(1, _D), relu=True)
    p2 = _sc_segment_sum(h, src, dst, zeros)
    out = _fused_linear(p2[:_N], p2[_N:], h, W2_rel, W2_root,
                        b2.reshape(1, _D), relu=False)
    return out
